# baseline (speedup 1.0000x reference)
import jax
import jax.numpy as jnp
from jax.experimental import pallas as pl
from jax.experimental.pallas import tpu as pltpu

B = 1024
F = 128
BITS = 8
ENC = F * BITS
H = 4096
C = 128
THR = 3.0
VMIN = 0.0
VMAX = 1.0

BH = 512
N = H // BH


def _fused_kernel(x_ref, w0_ref, w1a_ref, w1b_ref, outc_ref, preds_ref,
                  outact_ref, enc_ref, z2_ref, acc_ref):
    j = pl.program_id(0)

    @pl.when(j == 0)
    def _encode():
        xc = jnp.clip(x_ref[...], VMIN, VMAX)
        norm = (xc - VMIN) / (VMAX - VMIN)
        scaled = jnp.round(norm * (2 ** BITS - 1)).astype(jnp.int32)
        gray = scaled ^ (scaled >> 1)
        rowf = jax.lax.broadcasted_iota(jnp.int32, (F, ENC), 0)
        colf = jax.lax.broadcasted_iota(jnp.int32, (F, ENC), 1)
        sel = (colf // BITS == rowf).astype(jnp.float32)
        gexp = jnp.dot(gray.astype(jnp.float32), sel,
                       preferred_element_type=jnp.float32)
        bitpos = jax.lax.broadcasted_iota(jnp.int32, (B, ENC), 1) % BITS
        bits = (gexp.astype(jnp.int32) >> bitpos) & 1
        enc_ref[...] = bits.astype(jnp.float8_e4m3fn)
        acc_ref[...] = jnp.zeros((B, C), jnp.float32)
        z2_ref[...] = jnp.zeros((B, H), jnp.float32)

    w0 = w0_ref[...].astype(jnp.float8_e4m3fn)
    z1 = jax.lax.dot_general(enc_ref[...], w0, (((1,), (1,)), ((), ())),
                             preferred_element_type=jnp.float32)
    a0 = (z1 >= THR).astype(jnp.float8_e4m3fn)

    c0 = outc_ref[0, pl.ds(j * BH, BH), :]
    acc_ref[...] += jnp.dot(a0.astype(jnp.bfloat16), c0.astype(jnp.bfloat16),
                            preferred_element_type=jnp.float32)

    w1a = w1a_ref[...].astype(jnp.float8_e4m3fn)
    w1b = w1b_ref[...].astype(jnp.float8_e4m3fn)
    z2_ref[:, :H // 2] += jax.lax.dot_general(
        a0, w1a, (((1,), (1,)), ((), ())), preferred_element_type=jnp.float32)
    z2_ref[:, H // 2:] += jax.lax.dot_general(
        a0, w1b, (((1,), (1,)), ((), ())), preferred_element_type=jnp.float32)

    @pl.when(j == N - 1)
    def _finish():
        a1 = (z2_ref[...] >= THR).astype(jnp.bfloat16)
        out = acc_ref[...] + jnp.dot(a1, outc_ref[1].astype(jnp.bfloat16),
                                     preferred_element_type=jnp.float32)
        outact_ref[...] = out
        preds_ref[0, :] = jnp.argmax(out, axis=1).astype(jnp.int32)


def kernel(trainOrTest, x, y, W0, W1, outC):
    preds2, outAct = pl.pallas_call(
        _fused_kernel,
        grid=(N,),
        in_specs=[
            pl.BlockSpec((B, F), lambda j: (0, 0)),
            pl.BlockSpec((BH, ENC), lambda j: (j, 0)),
            pl.BlockSpec((H // 2, BH), lambda j: (0, j)),
            pl.BlockSpec((H // 2, BH), lambda j: (1, j)),
            pl.BlockSpec((2, H, C), lambda j: (0, 0, 0)),
        ],
        out_specs=[
            pl.BlockSpec((1, B), lambda j: (0, 0)),
            pl.BlockSpec((B, C), lambda j: (0, 0)),
        ],
        out_shape=[
            jax.ShapeDtypeStruct((1, B), jnp.int32),
            jax.ShapeDtypeStruct((B, C), jnp.float32),
        ],
        scratch_shapes=[
            pltpu.VMEM((B, ENC), jnp.float8_e4m3fn),
            pltpu.VMEM((B, H), jnp.float32),
            pltpu.VMEM((B, C), jnp.float32),
        ],
        compiler_params=pltpu.CompilerParams(
            dimension_semantics=("arbitrary",),
        ),
    )(x, W0, W1, W1, outC)
    return preds2[0], outAct


# speedup vs baseline: 1.0694x; 1.0694x over previous
import jax
import jax.numpy as jnp
from jax.experimental import pallas as pl
from jax.experimental.pallas import tpu as pltpu

B = 1024
F = 128
BITS = 8
ENC = F * BITS
H = 4096
C = 128
THR = 3.0
VMIN = 0.0
VMAX = 1.0

BH = 512
N = H // BH


def _fused_kernel(x_ref, w0_ref, w1_ref, outc_ref, preds_ref, outact_ref,
                  enc_ref, z2_ref, acc_ref, w1buf_ref, sem):
    j = pl.program_id(0)

    def w1_copy(k, slot):
        return pltpu.make_async_copy(
            w1_ref.at[:, pl.ds(k * BH, BH)], w1buf_ref.at[slot], sem.at[slot])

    @pl.when(j == 0)
    def _start():
        w1_copy(0, 0).start()
        w1_copy(1, 1).start()

    @pl.when(j == 0)
    def _encode():
        xc = jnp.clip(x_ref[...], VMIN, VMAX)
        norm = (xc - VMIN) / (VMAX - VMIN)
        scaled = jnp.round(norm * (2 ** BITS - 1)).astype(jnp.int32)
        gray = scaled ^ (scaled >> 1)
        rowf = jax.lax.broadcasted_iota(jnp.int32, (F, ENC), 0)
        colf = jax.lax.broadcasted_iota(jnp.int32, (F, ENC), 1)
        sel = (colf // BITS == rowf).astype(jnp.float32)
        gexp = jnp.dot(gray.astype(jnp.float32), sel,
                       preferred_element_type=jnp.float32)
        bitpos = jax.lax.broadcasted_iota(jnp.int32, (B, ENC), 1) % BITS
        bits = (gexp.astype(jnp.int32) >> bitpos) & 1
        enc_ref[...] = bits.astype(jnp.float8_e4m3fn)
        acc_ref[...] = jnp.zeros((B, C), jnp.float32)
        z2_ref[...] = jnp.zeros((B, H), jnp.float32)

    w0 = w0_ref[...].astype(jnp.float8_e4m3fn)
    z1 = jax.lax.dot_general(enc_ref[...], w0, (((1,), (1,)), ((), ())),
                             preferred_element_type=jnp.float32)
    a0 = (z1 >= THR).astype(jnp.float8_e4m3fn)

    c0 = outc_ref[0, pl.ds(j * BH, BH), :]
    acc_ref[...] += jnp.dot(a0.astype(jnp.bfloat16), c0.astype(jnp.bfloat16),
                            preferred_element_type=jnp.float32)

    @pl.when((j > 0) & (j < N - 1))
    def _prefetch():
        w1_copy(j + 1, (j + 1) % 2).start()

    w1_copy(j, j % 2).wait()
    slot = j % 2
    w1a = w1buf_ref[slot, :H // 2, :].astype(jnp.float8_e4m3fn)
    w1b = w1buf_ref[slot, H // 2:, :].astype(jnp.float8_e4m3fn)
    z2_ref[:, :H // 2] += jax.lax.dot_general(
        a0, w1a, (((1,), (1,)), ((), ())), preferred_element_type=jnp.float32)
    z2_ref[:, H // 2:] += jax.lax.dot_general(
        a0, w1b, (((1,), (1,)), ((), ())), preferred_element_type=jnp.float32)

    @pl.when(j == N - 1)
    def _finish():
        a1 = (z2_ref[...] >= THR).astype(jnp.bfloat16)
        out = acc_ref[...] + jnp.dot(a1, outc_ref[1].astype(jnp.bfloat16),
                                     preferred_element_type=jnp.float32)
        outact_ref[...] = out
        preds_ref[0, :] = jnp.argmax(out, axis=1).astype(jnp.int32)


def kernel(trainOrTest, x, y, W0, W1, outC):
    preds2, outAct = pl.pallas_call(
        _fused_kernel,
        grid=(N,),
        in_specs=[
            pl.BlockSpec((B, F), lambda j: (0, 0)),
            pl.BlockSpec((BH, ENC), lambda j: (j, 0)),
            pl.BlockSpec(memory_space=pl.ANY),
            pl.BlockSpec((2, H, C), lambda j: (0, 0, 0)),
        ],
        out_specs=[
            pl.BlockSpec((1, B), lambda j: (0, 0)),
            pl.BlockSpec((B, C), lambda j: (0, 0)),
        ],
        out_shape=[
            jax.ShapeDtypeStruct((1, B), jnp.int32),
            jax.ShapeDtypeStruct((B, C), jnp.float32),
        ],
        scratch_shapes=[
            pltpu.VMEM((B, ENC), jnp.float8_e4m3fn),
            pltpu.VMEM((B, H), jnp.float32),
            pltpu.VMEM((B, C), jnp.float32),
            pltpu.VMEM((2, H, BH), jnp.float32),
            pltpu.SemaphoreType.DMA((2,)),
        ],
        compiler_params=pltpu.CompilerParams(
            dimension_semantics=("arbitrary",),
        ),
    )(x, W0, W1, outC)
    return preds2[0], outAct
